# baseline (speedup 1.0000x reference)
import functools

import jax
import jax.numpy as jnp
from jax import lax
from jax.experimental import pallas as pl
from jax.experimental.pallas import tpu as pltpu
from jax.experimental.pallas import tpu_sc as plsc

E = 8
NG = 2
GS = E // NG
D = 1024
FF = 512
RSF = 2.5
T = 2048

TBG = 256
NCORE = 2
NSUB = 16
CHUNK = 64
THALF = T // NCORE
AHALF = 2 * THALF + E * TBG - E * 1

NBLK_HALF = 2 * THALF // TBG + E
AH = NBLK_HALF * TBG
NBLK3 = NCORE * NBLK_HALF
A = NCORE * AH


def _splat(x):
    return jax.lax.broadcast_in_dim(x, (16,), ())


def _cumsum16(x, lanes, zero):
    y = x
    for st in (1, 2, 4, 8):
        sh = y.at[jnp.maximum(lanes - st, zero * 0)].get(
            mode='promise_in_bounds')
        y = y + jnp.where(lanes >= (zero * 0 + st), sh, zero)
    return y


def _dot_t(a, b):
    return jax.lax.dot_general(a, b, (((1,), (1,)), ((), ())),
                               preferred_element_type=jnp.float32)


def _logits_body(gw_ref, x_ref, o_ref):
    o_ref[0] = jax.lax.dot_general(
        gw_ref[...], x_ref[...], (((1,), (1,)), ((), ())),
        preferred_element_type=jnp.float32)


def _logits(gw, x):
    nw = T // CHUNK
    return pl.pallas_call(
        _logits_body,
        grid=(nw,),
        in_specs=[
            pl.BlockSpec((E, D), lambda w: (0, 0)),
            pl.BlockSpec((CHUNK, D), lambda w: (w, 0)),
        ],
        out_specs=pl.BlockSpec((1, E, CHUNK), lambda w: (w, 0, 0)),
        out_shape=jax.ShapeDtypeStruct((nw, E, CHUNK), jnp.float32),
        compiler_params=pltpu.CompilerParams(
            dimension_semantics=("arbitrary",)),
    )(gw, x)


def _route_body(lt_hbm, b16_hbm, x_hbm,
                w_hbm, be_hbm, pos1_hbm, pos2_hbm, xs_hbm,
                lt_v, b_v, cnt_v, lc_v, pub_v, be_v,
                dst_v, tokv_v, wv_v, p1_v, p2_v,
                idx_v, rows_v, wrow_v, tok_sh, w_sh, counts_sh, pub_sh,
                sem, sem2, semw, tot_sm):
    cid = lax.axis_index("c")
    sid = lax.axis_index("s")
    tbase = cid * THALF + sid * CHUNK
    half = cid * AH

    wid = cid * NSUB + sid
    pltpu.sync_copy(lt_hbm.at[pl.ds(wid * (E * CHUNK), E * CHUNK)], lt_v)
    pltpu.sync_copy(b16_hbm, b_v)

    lanes = lax.broadcasted_iota(jnp.int32, (16,), 0)
    one_f = jnp.zeros((16,), jnp.float32) + 1.0
    negbig = jnp.zeros((16,), jnp.float32) - 1e30
    zero_f = jnp.zeros((16,), jnp.float32)
    one_i = jnp.zeros((16,), jnp.int32) + 1
    zero_i = jnp.zeros((16,), jnp.int32)
    nine_v = jnp.zeros((16,), jnp.int32) + (E + 1)
    e_vs = [jnp.zeros((16,), jnp.int32) + e for e in range(E)]

    e1s, e2s, w1s, w2s = [], [], [], []
    for j in range(CHUNK // 16):
        s = [1.0 / (1.0 + jnp.exp(-lt_v[pl.ds(e * CHUNK + j * 16, 16)]))
             for e in range(E)]
        sc = [s[e] + b_v[pl.ds(e * 16, 16)] for e in range(E)]

        two_i = one_i + one_i

        def top2sum(grp):
            m1 = jnp.maximum(jnp.maximum(grp[0], grp[1]),
                             jnp.maximum(grp[2], grp[3]))
            eqs = [g == m1 for g in grp]
            neq = zero_i
            for eq in eqs:
                neq = neq + jnp.where(eq, one_i, zero_i)
            m2c = [jnp.where(eq, negbig, g) for eq, g in zip(eqs, grp)]
            m2 = jnp.maximum(jnp.maximum(m2c[0], m2c[1]),
                             jnp.maximum(m2c[2], m2c[3]))
            m2 = jnp.where(neq >= two_i, m1, m2)
            return m1 + m2

        g0 = top2sum(sc[0:GS])
        g1 = top2sum(sc[GS:E])
        sel0 = g0 >= g1
        sel1 = g1 > g0
        v = [jnp.where(sel0 if e < GS else sel1, sc[e], zero_f)
             for e in range(E)]
        m1v = v[0]
        for e in range(1, E):
            m1v = jnp.maximum(m1v, v[e])
        idx1 = nine_v
        for e in range(E):
            idx1 = jnp.minimum(idx1, jnp.where(v[e] == m1v, e_vs[e],
                                               nine_v))
        w1 = jnp.zeros((16,), jnp.float32)
        for e in range(E):
            w1 = w1 + jnp.where(idx1 == e_vs[e], s[e], zero_f)
        v2 = [jnp.where(idx1 == e_vs[e], negbig, v[e]) for e in range(E)]
        m2v = v2[0]
        for e in range(1, E):
            m2v = jnp.maximum(m2v, v2[e])
        idx2 = nine_v
        for e in range(E):
            idx2 = jnp.minimum(idx2, jnp.where(v2[e] == m2v, e_vs[e],
                                               nine_v))
        w2 = jnp.zeros((16,), jnp.float32)
        for e in range(E):
            w2 = w2 + jnp.where(idx2 == e_vs[e], s[e], zero_f)
        scale = jnp.float32(RSF) / (w1 + w2 + jnp.float32(1e-20))
        e1s.append(idx1)
        e2s.append(idx2)
        w1s.append(w1 * scale)
        w2s.append(w2 * scale)

    cntrow = jnp.zeros((16,), jnp.int32)
    for e in range(E):
        cvec = jnp.zeros((16,), jnp.int32)
        for j in range(CHUNK // 16):
            cvec = (cvec + jnp.where(e1s[j] == e_vs[e], one_i, zero_i)
                    + jnp.where(e2s[j] == e_vs[e], one_i, zero_i))
        cntrow = cntrow + jnp.where(lanes == e_vs[e], cvec, zero_i)
    cnt_v[...] = cntrow + cntrow.at[zero_i + 15].get(mode='promise_in_bounds')
    pltpu.sync_copy(cnt_v, counts_sh.at[pl.ds(sid * 16, 16)])
    plsc.subcore_barrier()
    if True:
        return

    @pl.when(sid == 0)
    def _offsets():
        pltpu.sync_copy(counts_sh, lc_v)
        tot = jnp.zeros((16,), jnp.int32)
        for w in range(NSUB):
            tot = tot + lc_v[pl.ds(w * 16, 16)]
        padded = ((tot + (TBG - 1)) >> 8) << 8
        csum = _cumsum16(padded, lanes, zero_i)
        lbase = csum - padded

        blkv = lanes * TBG
        bev = jnp.zeros((16,), jnp.int32) + jnp.int32(E)
        for e in range(E):
            base_e = jnp.sum(jnp.where(lanes == e_vs[e], lbase, zero_i))
            pad_e = jnp.sum(jnp.where(lanes == e_vs[e], padded, zero_i))
            ind = jnp.logical_and(blkv >= _splat(base_e),
                                  blkv < _splat(base_e + pad_e))
            bev = jnp.where(ind, e_vs[e], bev)
        be_v[...] = bev
        pltpu.sync_copy(be_v, be_hbm.at[pl.ds(cid * NBLK_HALF, 16)])

        acc = lbase
        for w in range(NSUB):
            c = lc_v[pl.ds(w * 16, 16)]
            pub_v[pl.ds(w * 16, 16)] = acc
            acc = acc + c
        pub_v[pl.ds(NSUB * 16, 16)] = csum.at[zero_i + (E - 1)].get(
            mode='promise_in_bounds')
        pltpu.sync_copy(pub_v, pub_sh)

        acc = lbase
        for w in range(NSUB):
            c = lc_v[pl.ds(w * 16, 16)]
            pub_v[pl.ds(w * 16, 16)] = acc
            acc = acc + c
        pub_v[NSUB, :] = _splat(jnp.sum(padded))
        pltpu.sync_copy(pub_v, pub_sh)

    plsc.subcore_barrier()

    if True:
        return
    pltpu.sync_copy(pub_sh.at[pl.ds(sid * 16, 16)], cnt_v)
    mystart = cnt_v[...]
    offs = [jnp.sum(jnp.where(lanes == e_vs[e], mystart, zero_i))
            for e in range(E)]
    for j in range(CHUNK // 16):
        tokv = _splat(tbase + j * 16) + lanes
        for slot in range(2):
            ev = (e1s, e2s)[slot][j]
            wv = (w1s, w2s)[slot][j]
            p_v = (p1_v, p2_v)[slot]
            dst = jnp.zeros((16,), jnp.int32)
            for e in range(E):
                m = ev == e_vs[e]
                ranks = _cumsum16(jnp.where(m, one_i, zero_i), lanes, zero_i)
                dst = jnp.where(m, _splat(offs[e]) + ranks - 1, dst)
                offs[e] = offs[e] + jnp.max(ranks)
            pos = (slot * (CHUNK // 16) + j) * 16
            dst_v[pl.ds(pos, 16)] = dst
            tokv_v[pl.ds(pos, 16)] = tokv
            wv_v[pl.ds(pos, 16)] = wv
            p_v[pl.ds(j * 16, 16)] = dst + _splat(half)
    pltpu.sync_copy(tokv_v, tok_sh.at[dst_v])
    pltpu.sync_copy(wv_v, w_sh.at[dst_v])
    pltpu.sync_copy(p1_v, pos1_hbm.at[pl.ds(tbase, CHUNK)])
    pltpu.sync_copy(p2_v, pos2_hbm.at[pl.ds(tbase, CHUNK)])
    plsc.subcore_barrier()

    pltpu.sync_copy(pub_sh.at[NSUB], cnt_v)
    loc_tot = jnp.sum(jnp.where(lanes == zero_i, cnt_v[...], zero_i))
    pltpu.sync_copy(pub_sh.at[pl.ds(NSUB * 16, 16)], tot_sm)
    loc_tot = tot_sm[0]
    myslot = sid * TBG
    cpw = pltpu.async_copy(w_sh.at[pl.ds(myslot, TBG)], wrow_v, semw)
    pltpu.sync_copy(tok_sh.at[pl.ds(myslot, TBG)], idx_v)
    for q in range(TBG // 16):
        vq = idx_v[pl.ds(q * 16, 16)]
        idx_v[pl.ds(q * 16, 16)] = jnp.minimum(
            jnp.maximum(vq, zero_i), zero_i + (T - 1))
    cpw.wait()
    pltpu.sync_copy(wrow_v, w_hbm.at[pl.ds(half + myslot, TBG)])

    @pl.when(myslot < loc_tot)
    def _gather():
        nch = TBG // 32
        sems = (sem, sem2)
        nxt = pltpu.async_copy(x_hbm.at[idx_v.at[pl.ds(0, 32)]],
                               rows_v.at[0], sems[0])
        prev = [nxt]

        for k in range(nch):
            cur = prev[0]
            cur.wait()
            if k + 1 < nch:
                prev[0] = pltpu.async_copy(
                    x_hbm.at[idx_v.at[pl.ds((k + 1) * 32, 32)]],
                    rows_v.at[(k + 1) % 2], sems[(k + 1) % 2])
            pltpu.sync_copy(rows_v.at[k % 2],
                            xs_hbm.at[pl.ds(half + myslot + k * 32, 32)])


def _sc_mesh():
    return plsc.VectorSubcoreMesh(core_axis_name="c", subcore_axis_name="s",
                                  num_cores=NCORE, num_subcores=NSUB)


def _route(logits_t, b16, x):
    run = functools.partial(
        pl.kernel,
        out_type=[
            jax.ShapeDtypeStruct((A,), jnp.float32),
            jax.ShapeDtypeStruct((NBLK3,), jnp.int32),
            jax.ShapeDtypeStruct((T,), jnp.int32),
            jax.ShapeDtypeStruct((T,), jnp.int32),
            jax.ShapeDtypeStruct((A, D), jnp.float32),
        ],
        mesh=_sc_mesh(),
        scratch_types=[
            pltpu.VMEM((E * CHUNK,), jnp.float32),
            pltpu.VMEM((E * 16,), jnp.float32),
            pltpu.VMEM((16,), jnp.int32),
            pltpu.VMEM((NSUB * 16,), jnp.int32),
            pltpu.VMEM(((NSUB + 1) * 16,), jnp.int32),
            pltpu.VMEM((16,), jnp.int32),
            pltpu.VMEM((2 * CHUNK,), jnp.int32),
            pltpu.VMEM((2 * CHUNK,), jnp.int32),
            pltpu.VMEM((2 * CHUNK,), jnp.float32),
            pltpu.VMEM((CHUNK,), jnp.int32),
            pltpu.VMEM((CHUNK,), jnp.int32),
            pltpu.VMEM((TBG,), jnp.int32),
            pltpu.VMEM((2, 32, D), jnp.float32),
            pltpu.VMEM((TBG,), jnp.float32),
            pltpu.VMEM_SHARED((AH,), jnp.int32),
            pltpu.VMEM_SHARED((AH,), jnp.float32),
            pltpu.VMEM_SHARED((NSUB * 16,), jnp.int32),
            pltpu.VMEM_SHARED(((NSUB + 1) * 16,), jnp.int32),
            pltpu.SemaphoreType.DMA,
            pltpu.SemaphoreType.DMA,
            pltpu.SemaphoreType.DMA,
            pltpu.SMEM((16,), jnp.int32),
        ],
    )(_route_body)
    return run(logits_t, b16, x)


def _shared_body(x_ref, sg_ref, su_ref, sd_ref, o_ref):
    xb = x_ref[...].astype(jnp.bfloat16)
    hg = _dot_t(xb, sg_ref[...].astype(jnp.bfloat16))
    hu = _dot_t(xb, su_ref[...].astype(jnp.bfloat16))
    h = (jax.nn.silu(hg) * hu).astype(jnp.bfloat16)
    o_ref[...] = _dot_t(h, sd_ref[...].astype(jnp.bfloat16))


def _shared(x, sg, su, sd):
    nblk = T // TBG
    return pl.pallas_call(
        _shared_body,
        grid=(nblk,),
        in_specs=[
            pl.BlockSpec((TBG, D), lambda i: (i, 0)),
            pl.BlockSpec((FF, D), lambda i: (0, 0)),
            pl.BlockSpec((FF, D), lambda i: (0, 0)),
            pl.BlockSpec((D, FF), lambda i: (0, 0)),
        ],
        out_specs=pl.BlockSpec((TBG, D), lambda i: (i, 0)),
        out_shape=jax.ShapeDtypeStruct((T, D), jnp.float32),
        compiler_params=pltpu.CompilerParams(
            dimension_semantics=("arbitrary",)),
    )(x, sg, su, sd)


def _group_body(be_ref, xs_ref, w_ref, eg_ref, eu_ref, ed_ref, y_ref):
    i = pl.program_id(0)
    be = be_ref[i]

    @pl.when(be < E)
    def _():
        xb = xs_ref[...].astype(jnp.bfloat16)
        hg = _dot_t(xb, eg_ref[0].astype(jnp.bfloat16))
        hu = _dot_t(xb, eu_ref[0].astype(jnp.bfloat16))
        h = jax.nn.silu(hg) * hu
        ri = lax.broadcasted_iota(jnp.int32, (TBG, TBG), 0)
        ci = lax.broadcasted_iota(jnp.int32, (TBG, TBG), 1)
        diag = jnp.where(ri == ci,
                         jnp.broadcast_to(w_ref[0], (TBG, TBG)), 0.0)
        hw = jax.lax.dot_general(diag, h, (((1,), (0,)), ((), ())),
                                 preferred_element_type=jnp.float32)
        y_ref[...] = _dot_t(hw.astype(jnp.bfloat16),
                            ed_ref[0].astype(jnp.bfloat16))


def _grouped(be, xs, w3, eg, eu, ed):
    grid_spec = pltpu.PrefetchScalarGridSpec(
        num_scalar_prefetch=1,
        grid=(NBLK3,),
        in_specs=[
            pl.BlockSpec((TBG, D), lambda i, b: (i, 0)),
            pl.BlockSpec((1, 1, TBG), lambda i, b: (i, 0, 0)),
            pl.BlockSpec((1, FF, D),
                         lambda i, b: (jnp.minimum(b[i], E - 1), 0, 0)),
            pl.BlockSpec((1, FF, D),
                         lambda i, b: (jnp.minimum(b[i], E - 1), 0, 0)),
            pl.BlockSpec((1, D, FF),
                         lambda i, b: (jnp.minimum(b[i], E - 1), 0, 0)),
        ],
        out_specs=pl.BlockSpec((TBG, D), lambda i, b: (i, 0)),
    )
    return pl.pallas_call(
        _group_body,
        grid_spec=grid_spec,
        out_shape=jax.ShapeDtypeStruct((A, D), jnp.float32),
        compiler_params=pltpu.CompilerParams(
            dimension_semantics=("arbitrary",)),
    )(be, xs, w3, eg, eu, ed)


def _combine_body(sh_hbm, y_hbm, pos1_hbm, pos2_hbm, out_hbm,
                  i1_v, i2_v, acc_v, r1_v, r2_v,
                  s10, s11, s20, s21, sacc):
    cid = lax.axis_index("c")
    sid = lax.axis_index("s")
    tbase = cid * THALF + sid * CHUNK
    pltpu.sync_copy(pos1_hbm.at[pl.ds(tbase, CHUNK)], i1_v)
    pltpu.sync_copy(pos2_hbm.at[pl.ds(tbase, CHUNK)], i2_v)
    s1s = (s10, s11)
    s2s = (s20, s21)
    nch = CHUNK // 16

    def issue(c):
        b = c % 2
        g1 = pltpu.async_copy(y_hbm.at[i1_v.at[pl.ds(c * 16, 16)]],
                              r1_v.at[b], s1s[b])
        g2 = pltpu.async_copy(y_hbm.at[i2_v.at[pl.ds(c * 16, 16)]],
                              r2_v.at[b], s2s[b])
        return g1, g2

    nxt = issue(0)
    for c in range(nch):
        tb = tbase + c * 16
        g1, g2 = nxt
        ga = pltpu.async_copy(sh_hbm.at[pl.ds(tb, 16), :], acc_v, sacc)
        g1.wait()
        g2.wait()
        ga.wait()
        if c + 1 < nch:
            nxt = issue(c + 1)
        b = c % 2

        def row_body(r, carry):
            for k in range(D // 16):
                slk = pl.ds(k * 16, 16)
                acc_v[r, slk] = (acc_v[r, slk] + r1_v[b, r, slk]
                                 + r2_v[b, r, slk])
            return carry

        lax.fori_loop(0, 16, row_body, 0)
        pltpu.sync_copy(acc_v, out_hbm.at[pl.ds(tb, 16), :])


def _combine(shared, y, pos1, pos2):
    run = functools.partial(
        pl.kernel,
        out_type=[jax.ShapeDtypeStruct((T, D), jnp.float32)],
        mesh=_sc_mesh(),
        scratch_types=[
            pltpu.VMEM((CHUNK,), jnp.int32),
            pltpu.VMEM((CHUNK,), jnp.int32),
            pltpu.VMEM((16, D), jnp.float32),
            pltpu.VMEM((2, 16, D), jnp.float32),
            pltpu.VMEM((2, 16, D), jnp.float32),
            pltpu.SemaphoreType.DMA,
            pltpu.SemaphoreType.DMA,
            pltpu.SemaphoreType.DMA,
            pltpu.SemaphoreType.DMA,
            pltpu.SemaphoreType.DMA,
        ],
    )(_combine_body)
    return run(shared, y, pos1, pos2)


@jax.jit
def _moe(x, gate_weight, bias, eg, eu, ed, sg, su, sd):
    logits_t = _logits(gate_weight, x)
    b16 = jnp.broadcast_to(bias.reshape(E, 1), (E, 16)).reshape(E * 16)
    wsort, be, pos1, pos2, xs = _route(logits_t.reshape(-1), b16, x)
    shared = _shared(x, sg, su, sd)
    y = _grouped(be, xs, wsort.reshape(NBLK3, 1, TBG), eg, eu, ed)
    (out,) = _combine(shared, y, pos1, pos2)
    return out


def kernel(hidden_states, gate_weight, e_score_correction_bias,
           expert_gate_w, expert_up_w, expert_down_w,
           shared_gate_w, shared_up_w, shared_down_w):
    orig_shape = hidden_states.shape
    x = hidden_states.reshape(-1, D).astype(jnp.float32)
    out = _moe(x, gate_weight, e_score_correction_bias,
               expert_gate_w, expert_up_w, expert_down_w,
               shared_gate_w, shared_up_w, shared_down_w)
    return out.reshape(orig_shape)


# speedup vs baseline: 1.2032x; 1.2032x over previous
import functools

import jax
import jax.numpy as jnp
from jax import lax
from jax.experimental import pallas as pl
from jax.experimental.pallas import tpu as pltpu
from jax.experimental.pallas import tpu_sc as plsc

E = 8
NG = 2
GS = E // NG
D = 1024
FF = 512
RSF = 2.5
T = 2048

TB = 256
NBLK = T // TB
NCORE = 2
NSUB = 16
CHUNK = 64
THALF = T // NCORE


def _dot_t(a, b):
    return jax.lax.dot_general(a, b, (((1,), (1,)), ((), ())),
                               preferred_element_type=jnp.float32)


def _sc_mesh():
    return plsc.VectorSubcoreMesh(core_axis_name="c", subcore_axis_name="s",
                                  num_cores=NCORE, num_subcores=NSUB)


def _logits_body(gw_ref, x_ref, o_ref):
    o_ref[0] = jax.lax.dot_general(
        gw_ref[...], x_ref[...], (((1,), (1,)), ((), ())),
        preferred_element_type=jnp.float32)


def _logits(gw, x):
    nw = T // CHUNK
    return pl.pallas_call(
        _logits_body,
        grid=(nw,),
        in_specs=[
            pl.BlockSpec((E, D), lambda w: (0, 0)),
            pl.BlockSpec((CHUNK, D), lambda w: (w, 0)),
        ],
        out_specs=pl.BlockSpec((1, E, CHUNK), lambda w: (w, 0, 0)),
        out_shape=jax.ShapeDtypeStruct((nw, E, CHUNK), jnp.float32),
        compiler_params=pltpu.CompilerParams(
            dimension_semantics=("arbitrary",)),
    )(gw, x)


def _route_body(lt_hbm, b16_hbm, combt_hbm, lt_v, b_v, comb_v):
    cid = lax.axis_index("c")
    sid = lax.axis_index("s")
    wid = cid * NSUB + sid
    tbase = wid * CHUNK

    pltpu.sync_copy(lt_hbm.at[pl.ds(wid * (E * CHUNK), E * CHUNK)], lt_v)
    pltpu.sync_copy(b16_hbm, b_v)

    negbig = jnp.zeros((16,), jnp.float32) - 1e30
    zero_f = jnp.zeros((16,), jnp.float32)
    zero_i = jnp.zeros((16,), jnp.int32)
    nine_v = jnp.zeros((16,), jnp.int32) + (E + 1)
    e_vs = [jnp.zeros((16,), jnp.int32) + e for e in range(E)]
    one_i = jnp.zeros((16,), jnp.int32) + 1
    two_i = one_i + one_i

    for j in range(CHUNK // 16):
        s = [1.0 / (1.0 + jnp.exp(-lt_v[pl.ds(e * CHUNK + j * 16, 16)]))
             for e in range(E)]
        sc = [s[e] + b_v[pl.ds(e * 16, 16)] for e in range(E)]

        def top2sum(grp):
            m1 = jnp.maximum(jnp.maximum(grp[0], grp[1]),
                             jnp.maximum(grp[2], grp[3]))
            eqs = [g == m1 for g in grp]
            neq = zero_i
            for eq in eqs:
                neq = neq + jnp.where(eq, one_i, zero_i)
            m2c = [jnp.where(eq, negbig, g) for eq, g in zip(eqs, grp)]
            m2 = jnp.maximum(jnp.maximum(m2c[0], m2c[1]),
                             jnp.maximum(m2c[2], m2c[3]))
            m2 = jnp.where(neq >= two_i, m1, m2)
            return m1 + m2

        g0 = top2sum(sc[0:GS])
        g1 = top2sum(sc[GS:E])
        sel0 = g0 >= g1
        sel1 = g1 > g0
        v = [jnp.where(sel0 if e < GS else sel1, sc[e], zero_f)
             for e in range(E)]
        m1v = v[0]
        for e in range(1, E):
            m1v = jnp.maximum(m1v, v[e])
        idx1 = nine_v
        for e in range(E):
            idx1 = jnp.minimum(idx1, jnp.where(v[e] == m1v, e_vs[e], nine_v))
        w1 = zero_f
        for e in range(E):
            w1 = w1 + jnp.where(idx1 == e_vs[e], s[e], zero_f)
        v2 = [jnp.where(idx1 == e_vs[e], negbig, v[e]) for e in range(E)]
        m2v = v2[0]
        for e in range(1, E):
            m2v = jnp.maximum(m2v, v2[e])
        idx2 = nine_v
        for e in range(E):
            idx2 = jnp.minimum(idx2, jnp.where(v2[e] == m2v, e_vs[e],
                                               nine_v))
        w2 = zero_f
        for e in range(E):
            w2 = w2 + jnp.where(idx2 == e_vs[e], s[e], zero_f)
        scale = jnp.float32(RSF) / (w1 + w2 + jnp.float32(1e-20))
        w1f = w1 * scale
        w2f = w2 * scale
        for e in range(E):
            ce = (jnp.where(idx1 == e_vs[e], w1f, zero_f)
                  + jnp.where(idx2 == e_vs[e], w2f, zero_f))
            comb_v[pl.ds(e * CHUNK + j * 16, 16)] = ce

    for e in range(E):
        pltpu.sync_copy(comb_v.at[pl.ds(e * CHUNK, CHUNK)],
                        combt_hbm.at[pl.ds(e * T + tbase, CHUNK)])


def _route(logits_flat, b16):
    run = functools.partial(
        pl.kernel,
        out_type=[jax.ShapeDtypeStruct((E * T,), jnp.float32)],
        mesh=_sc_mesh(),
        scratch_types=[
            pltpu.VMEM((E * CHUNK,), jnp.float32),
            pltpu.VMEM((E * 16,), jnp.float32),
            pltpu.VMEM((E * CHUNK,), jnp.float32),
        ],
    )(_route_body)
    (combt,) = run(logits_flat, b16)
    return combt


def _transpose_body(a_ref, o_ref):
    o_ref[...] = a_ref[...].T


def _transpose(a):
    return pl.pallas_call(
        _transpose_body,
        out_shape=jax.ShapeDtypeStruct((T, E), jnp.float32),
    )(a)


def _moe_body(x_ref, comb_ref, eg_ref, eu_ref, ed_ref,
              sg_ref, su_ref, sd_ref, o_ref, acc_ref, xb_ref):
    e = pl.program_id(0)
    i = pl.program_id(1)
    rows = pl.ds(i * TB, TB)

    @pl.when(e == 0)
    def _init():
        xb = x_ref[rows, :].astype(jnp.bfloat16)
        xb_ref[rows, :] = xb
        hg = _dot_t(xb, sg_ref[...].astype(jnp.bfloat16))
        hu = _dot_t(xb, su_ref[...].astype(jnp.bfloat16))
        h = (jax.nn.silu(hg) * hu).astype(jnp.bfloat16)
        acc_ref[rows, :] = _dot_t(h, sd_ref[...].astype(jnp.bfloat16))

    xb = xb_ref[rows, :]
    hg = _dot_t(xb, eg_ref[0].astype(jnp.bfloat16))
    hu = _dot_t(xb, eu_ref[0].astype(jnp.bfloat16))
    h = (jax.nn.silu(hg) * hu).astype(jnp.bfloat16)
    eo = _dot_t(h, ed_ref[0].astype(jnp.bfloat16))
    cols = jax.lax.broadcasted_iota(jnp.int32, (TB, E), 1)
    ce = jnp.sum(jnp.where(cols == e, comb_ref[rows, :], 0.0), axis=1,
                 keepdims=True)
    acc_ref[rows, :] = acc_ref[rows, :] + eo * ce

    @pl.when(e == E - 1)
    def _fin():
        o_ref[...] = acc_ref[rows, :]


def _moe_dense(x, comb, eg, eu, ed, sg, su, sd):
    grid = (E, NBLK)
    return pl.pallas_call(
        _moe_body,
        grid=grid,
        in_specs=[
            pl.BlockSpec((T, D), lambda e, i: (0, 0)),
            pl.BlockSpec((T, E), lambda e, i: (0, 0)),
            pl.BlockSpec((1, FF, D), lambda e, i: (e, 0, 0)),
            pl.BlockSpec((1, FF, D), lambda e, i: (e, 0, 0)),
            pl.BlockSpec((1, D, FF), lambda e, i: (e, 0, 0)),
            pl.BlockSpec((FF, D), lambda e, i: (0, 0)),
            pl.BlockSpec((FF, D), lambda e, i: (0, 0)),
            pl.BlockSpec((D, FF), lambda e, i: (0, 0)),
        ],
        out_specs=pl.BlockSpec(
            (TB, D), lambda e, i: (jnp.where(e == E - 1, i, 0), 0)),
        out_shape=jax.ShapeDtypeStruct((T, D), jnp.float32),
        scratch_shapes=[
            pltpu.VMEM((T, D), jnp.float32),
            pltpu.VMEM((T, D), jnp.bfloat16),
        ],
        compiler_params=pltpu.CompilerParams(
            dimension_semantics=("arbitrary", "arbitrary"),
        ),
    )(x, comb, eg, eu, ed, sg, su, sd)


@jax.jit
def _moe(x, gate_weight, bias, eg, eu, ed, sg, su, sd):
    logits = _logits(gate_weight, x)
    b16 = jnp.broadcast_to(bias.reshape(E, 1), (E, 16)).reshape(E * 16)
    combt = _route(logits.reshape(-1), b16)
    comb = _transpose(combt.reshape(E, T))
    return _moe_dense(x, comb, eg, eu, ed, sg, su, sd)


def kernel(hidden_states, gate_weight, e_score_correction_bias,
           expert_gate_w, expert_up_w, expert_down_w,
           shared_gate_w, shared_up_w, shared_down_w):
    orig_shape = hidden_states.shape
    x = hidden_states.reshape(-1, D).astype(jnp.float32)
    out = _moe(x, gate_weight, e_score_correction_bias,
               expert_gate_w, expert_up_w, expert_down_w,
               shared_gate_w, shared_up_w, shared_down_w)
    return out.reshape(orig_shape)


# speedup vs baseline: 1.3173x; 1.0949x over previous
import functools

import jax
import jax.numpy as jnp
from jax import lax
from jax.experimental import pallas as pl
from jax.experimental.pallas import tpu as pltpu
from jax.experimental.pallas import tpu_sc as plsc

E = 8
NG = 2
GS = E // NG
D = 1024
FF = 512
RSF = 2.5
T = 2048

TB = 256
NBLK = T // TB
NCORE = 2
NSUB = 16
CHUNK = 64
THALF = T // NCORE


def _dot_t(a, b):
    return jax.lax.dot_general(a, b, (((1,), (1,)), ((), ())),
                               preferred_element_type=jnp.float32)


def _sc_mesh():
    return plsc.VectorSubcoreMesh(core_axis_name="c", subcore_axis_name="s",
                                  num_cores=NCORE, num_subcores=NSUB)


def _logits_body(gw_ref, x_ref, o_ref):
    o_ref[...] = jax.lax.dot_general(
        gw_ref[...], x_ref[...], (((1,), (1,)), ((), ())),
        preferred_element_type=jnp.float32)


def _logits(gw, x):
    return pl.pallas_call(
        _logits_body,
        out_shape=jax.ShapeDtypeStruct((E, T), jnp.float32),
    )(gw, x)


def _route_body(lt_hbm, b16_hbm, combt_hbm, lt_v, b_v, comb_v):
    cid = lax.axis_index("c")
    sid = lax.axis_index("s")
    wid = cid * NSUB + sid
    tbase = wid * CHUNK

    for e in range(E):
        pltpu.sync_copy(lt_hbm.at[pl.ds(e * T + tbase, CHUNK)],
                        lt_v.at[pl.ds(e * CHUNK, CHUNK)])
    pltpu.sync_copy(b16_hbm, b_v)

    negbig = jnp.zeros((16,), jnp.float32) - 1e30
    zero_f = jnp.zeros((16,), jnp.float32)
    zero_i = jnp.zeros((16,), jnp.int32)
    nine_v = jnp.zeros((16,), jnp.int32) + (E + 1)
    e_vs = [jnp.zeros((16,), jnp.int32) + e for e in range(E)]
    one_i = jnp.zeros((16,), jnp.int32) + 1
    two_i = one_i + one_i

    for j in range(CHUNK // 16):
        s = [1.0 / (1.0 + jnp.exp(-lt_v[pl.ds(e * CHUNK + j * 16, 16)]))
             for e in range(E)]
        sc = [s[e] + b_v[pl.ds(e * 16, 16)] for e in range(E)]

        def top2sum(grp):
            m1 = jnp.maximum(jnp.maximum(grp[0], grp[1]),
                             jnp.maximum(grp[2], grp[3]))
            eqs = [g == m1 for g in grp]
            neq = zero_i
            for eq in eqs:
                neq = neq + jnp.where(eq, one_i, zero_i)
            m2c = [jnp.where(eq, negbig, g) for eq, g in zip(eqs, grp)]
            m2 = jnp.maximum(jnp.maximum(m2c[0], m2c[1]),
                             jnp.maximum(m2c[2], m2c[3]))
            m2 = jnp.where(neq >= two_i, m1, m2)
            return m1 + m2

        g0 = top2sum(sc[0:GS])
        g1 = top2sum(sc[GS:E])
        sel0 = g0 >= g1
        sel1 = g1 > g0
        v = [jnp.where(sel0 if e < GS else sel1, sc[e], zero_f)
             for e in range(E)]
        m1v = v[0]
        for e in range(1, E):
            m1v = jnp.maximum(m1v, v[e])
        idx1 = nine_v
        for e in range(E):
            idx1 = jnp.minimum(idx1, jnp.where(v[e] == m1v, e_vs[e], nine_v))
        w1 = zero_f
        for e in range(E):
            w1 = w1 + jnp.where(idx1 == e_vs[e], s[e], zero_f)
        v2 = [jnp.where(idx1 == e_vs[e], negbig, v[e]) for e in range(E)]
        m2v = v2[0]
        for e in range(1, E):
            m2v = jnp.maximum(m2v, v2[e])
        idx2 = nine_v
        for e in range(E):
            idx2 = jnp.minimum(idx2, jnp.where(v2[e] == m2v, e_vs[e],
                                               nine_v))
        w2 = zero_f
        for e in range(E):
            w2 = w2 + jnp.where(idx2 == e_vs[e], s[e], zero_f)
        scale = jnp.float32(RSF) / (w1 + w2 + jnp.float32(1e-20))
        w1f = w1 * scale
        w2f = w2 * scale
        for e in range(E):
            ce = (jnp.where(idx1 == e_vs[e], w1f, zero_f)
                  + jnp.where(idx2 == e_vs[e], w2f, zero_f))
            comb_v[pl.ds(e * CHUNK + j * 16, 16)] = ce

    for e in range(E):
        pltpu.sync_copy(comb_v.at[pl.ds(e * CHUNK, CHUNK)],
                        combt_hbm.at[pl.ds(e * T + tbase, CHUNK)])


def _route(logits_flat, b16):
    run = functools.partial(
        pl.kernel,
        out_type=[jax.ShapeDtypeStruct((E * T,), jnp.float32)],
        mesh=_sc_mesh(),
        scratch_types=[
            pltpu.VMEM((E * CHUNK,), jnp.float32),
            pltpu.VMEM((E * 16,), jnp.float32),
            pltpu.VMEM((E * CHUNK,), jnp.float32),
        ],
    )(_route_body)
    (combt,) = run(logits_flat, b16)
    return combt


def _moe_body(x_ref, combt_ref, eg_ref, eu_ref, ed_ref,
              sg_ref, su_ref, sd_ref, o_ref, acc_ref, xb_ref, comb_ref):
    e = pl.program_id(0)
    i = pl.program_id(1)
    rows = pl.ds(i * TB, TB)

    @pl.when(e == 0)
    def _init():
        @pl.when(i == 0)
        def _tr():
            comb_ref[...] = combt_ref[...].T

        xb = x_ref[rows, :].astype(jnp.bfloat16)
        xb_ref[rows, :] = xb
        hg = _dot_t(xb, sg_ref[...].astype(jnp.bfloat16))
        hu = _dot_t(xb, su_ref[...].astype(jnp.bfloat16))
        h = (jax.nn.silu(hg) * hu).astype(jnp.bfloat16)
        acc_ref[rows, :] = _dot_t(h, sd_ref[...].astype(jnp.bfloat16))

    xb = xb_ref[rows, :]
    hg = _dot_t(xb, eg_ref[0].astype(jnp.bfloat16))
    hu = _dot_t(xb, eu_ref[0].astype(jnp.bfloat16))
    h = (jax.nn.silu(hg) * hu).astype(jnp.bfloat16)
    eo = _dot_t(h, ed_ref[0].astype(jnp.bfloat16))
    cols = jax.lax.broadcasted_iota(jnp.int32, (TB, E), 1)
    ce = jnp.sum(jnp.where(cols == e, comb_ref[rows, :], 0.0), axis=1,
                 keepdims=True)
    acc_ref[rows, :] = acc_ref[rows, :] + eo * ce

    @pl.when(e == E - 1)
    def _fin():
        o_ref[...] = acc_ref[rows, :]


def _moe_dense(x, combt, eg, eu, ed, sg, su, sd):
    grid = (E, NBLK)
    return pl.pallas_call(
        _moe_body,
        grid=grid,
        in_specs=[
            pl.BlockSpec((T, D), lambda e, i: (0, 0)),
            pl.BlockSpec((E, T), lambda e, i: (0, 0)),
            pl.BlockSpec((1, FF, D), lambda e, i: (e, 0, 0)),
            pl.BlockSpec((1, FF, D), lambda e, i: (e, 0, 0)),
            pl.BlockSpec((1, D, FF), lambda e, i: (e, 0, 0)),
            pl.BlockSpec((FF, D), lambda e, i: (0, 0)),
            pl.BlockSpec((FF, D), lambda e, i: (0, 0)),
            pl.BlockSpec((D, FF), lambda e, i: (0, 0)),
        ],
        out_specs=pl.BlockSpec(
            (TB, D), lambda e, i: (jnp.where(e == E - 1, i, 0), 0)),
        out_shape=jax.ShapeDtypeStruct((T, D), jnp.float32),
        scratch_shapes=[
            pltpu.VMEM((T, D), jnp.float32),
            pltpu.VMEM((T, D), jnp.bfloat16),
            pltpu.VMEM((T, E), jnp.float32),
        ],
        compiler_params=pltpu.CompilerParams(
            dimension_semantics=("arbitrary", "arbitrary"),
        ),
    )(x, combt, eg, eu, ed, sg, su, sd)


@jax.jit
def _moe(x, gate_weight, bias, eg, eu, ed, sg, su, sd):
    logits = _logits(gate_weight, x)
    b16 = jnp.broadcast_to(bias.reshape(E, 1), (E, 16)).reshape(E * 16)
    combt = _route(logits.reshape(-1), b16)
    return _moe_dense(x, combt.reshape(E, T), eg, eu, ed, sg, su, sd)


def kernel(hidden_states, gate_weight, e_score_correction_bias,
           expert_gate_w, expert_up_w, expert_down_w,
           shared_gate_w, shared_up_w, shared_down_w):
    orig_shape = hidden_states.shape
    x = hidden_states.reshape(-1, D).astype(jnp.float32)
    out = _moe(x, gate_weight, e_score_correction_bias,
               expert_gate_w, expert_up_w, expert_down_w,
               shared_gate_w, shared_up_w, shared_down_w)
    return out.reshape(orig_shape)


# speedup vs baseline: 1.3380x; 1.0157x over previous
import functools

import jax
import jax.numpy as jnp
from jax import lax
from jax.experimental import pallas as pl
from jax.experimental.pallas import tpu as pltpu
from jax.experimental.pallas import tpu_sc as plsc

E = 8
NG = 2
GS = E // NG
D = 1024
FF = 512
RSF = 2.5
T = 2048

TB = 256
NBLK = T // TB
NCORE = 2
NSUB = 16
CHUNK = 64
THALF = T // NCORE


def _dot_t(a, b):
    return jax.lax.dot_general(a, b, (((1,), (1,)), ((), ())),
                               preferred_element_type=jnp.float32)


def _sc_mesh():
    return plsc.VectorSubcoreMesh(core_axis_name="c", subcore_axis_name="s",
                                  num_cores=NCORE, num_subcores=NSUB)


def _logits_body(gw_ref, x_ref, o_ref):
    lt = jax.lax.dot_general(
        gw_ref[...], x_ref[...], (((1,), (1,)), ((), ())),
        preferred_element_type=jnp.float32)
    nw = T // CHUNK
    o_ref[...] = jnp.transpose(lt.reshape(E, nw, CHUNK), (1, 0, 2))


def _logits(gw, x):
    nw = T // CHUNK
    return pl.pallas_call(
        _logits_body,
        out_shape=jax.ShapeDtypeStruct((nw, E, CHUNK), jnp.float32),
    )(gw, x)


def _route_body(lt_hbm, b16_hbm, combt_hbm, lt_v, b_v, comb_v):
    cid = lax.axis_index("c")
    sid = lax.axis_index("s")
    wid = cid * NSUB + sid
    tbase = wid * CHUNK

    pltpu.sync_copy(lt_hbm.at[pl.ds(wid * (E * CHUNK), E * CHUNK)], lt_v)
    pltpu.sync_copy(b16_hbm, b_v)

    negbig = jnp.zeros((16,), jnp.float32) - 1e30
    zero_f = jnp.zeros((16,), jnp.float32)
    zero_i = jnp.zeros((16,), jnp.int32)
    nine_v = jnp.zeros((16,), jnp.int32) + (E + 1)
    e_vs = [jnp.zeros((16,), jnp.int32) + e for e in range(E)]
    one_i = jnp.zeros((16,), jnp.int32) + 1
    two_i = one_i + one_i

    for j in range(CHUNK // 16):
        s = [1.0 / (1.0 + jnp.exp(-lt_v[pl.ds(e * CHUNK + j * 16, 16)]))
             for e in range(E)]
        sc = [s[e] + b_v[pl.ds(e * 16, 16)] for e in range(E)]

        def top2sum(grp):
            m1 = jnp.maximum(jnp.maximum(grp[0], grp[1]),
                             jnp.maximum(grp[2], grp[3]))
            eqs = [g == m1 for g in grp]
            neq = zero_i
            for eq in eqs:
                neq = neq + jnp.where(eq, one_i, zero_i)
            m2c = [jnp.where(eq, negbig, g) for eq, g in zip(eqs, grp)]
            m2 = jnp.maximum(jnp.maximum(m2c[0], m2c[1]),
                             jnp.maximum(m2c[2], m2c[3]))
            m2 = jnp.where(neq >= two_i, m1, m2)
            return m1 + m2

        g0 = top2sum(sc[0:GS])
        g1 = top2sum(sc[GS:E])
        sel0 = g0 >= g1
        sel1 = g1 > g0
        v = [jnp.where(sel0 if e < GS else sel1, sc[e], zero_f)
             for e in range(E)]
        m1v = v[0]
        for e in range(1, E):
            m1v = jnp.maximum(m1v, v[e])
        idx1 = nine_v
        for e in range(E):
            idx1 = jnp.minimum(idx1, jnp.where(v[e] == m1v, e_vs[e], nine_v))
        w1 = zero_f
        for e in range(E):
            w1 = w1 + jnp.where(idx1 == e_vs[e], s[e], zero_f)
        v2 = [jnp.where(idx1 == e_vs[e], negbig, v[e]) for e in range(E)]
        m2v = v2[0]
        for e in range(1, E):
            m2v = jnp.maximum(m2v, v2[e])
        idx2 = nine_v
        for e in range(E):
            idx2 = jnp.minimum(idx2, jnp.where(v2[e] == m2v, e_vs[e],
                                               nine_v))
        w2 = zero_f
        for e in range(E):
            w2 = w2 + jnp.where(idx2 == e_vs[e], s[e], zero_f)
        scale = jnp.float32(RSF) / (w1 + w2 + jnp.float32(1e-20))
        w1f = w1 * scale
        w2f = w2 * scale
        for e in range(E):
            ce = (jnp.where(idx1 == e_vs[e], w1f, zero_f)
                  + jnp.where(idx2 == e_vs[e], w2f, zero_f))
            comb_v[pl.ds(e * CHUNK + j * 16, 16)] = ce

    for e in range(E):
        pltpu.sync_copy(comb_v.at[pl.ds(e * CHUNK, CHUNK)],
                        combt_hbm.at[pl.ds(e * T + tbase, CHUNK)])


def _route(logits_flat, b16):
    run = functools.partial(
        pl.kernel,
        out_type=[jax.ShapeDtypeStruct((E * T,), jnp.float32)],
        mesh=_sc_mesh(),
        scratch_types=[
            pltpu.VMEM((E * CHUNK,), jnp.float32),
            pltpu.VMEM((E * 16,), jnp.float32),
            pltpu.VMEM((E * CHUNK,), jnp.float32),
        ],
    )(_route_body)
    (combt,) = run(logits_flat, b16)
    return combt


def _moe_body(x_ref, combt_ref, eg_ref, eu_ref, ed_ref,
              sg_ref, su_ref, sd_ref, o_ref, acc_ref, xb_ref, comb_ref):
    e = pl.program_id(0)
    i = pl.program_id(1)
    rows = pl.ds(i * TB, TB)

    @pl.when(e == 0)
    def _init():
        @pl.when(i == 0)
        def _tr():
            comb_ref[...] = combt_ref[...].T

        xb = x_ref[rows, :].astype(jnp.bfloat16)
        xb_ref[rows, :] = xb
        hg = _dot_t(xb, sg_ref[...].astype(jnp.bfloat16))
        hu = _dot_t(xb, su_ref[...].astype(jnp.bfloat16))
        h = (jax.nn.silu(hg) * hu).astype(jnp.bfloat16)
        acc_ref[rows, :] = _dot_t(h, sd_ref[...].astype(jnp.bfloat16))

    xb = xb_ref[rows, :]
    hg = _dot_t(xb, eg_ref[0].astype(jnp.bfloat16))
    hu = _dot_t(xb, eu_ref[0].astype(jnp.bfloat16))
    h = (jax.nn.silu(hg) * hu).astype(jnp.bfloat16)
    eo = _dot_t(h, ed_ref[0].astype(jnp.bfloat16))
    cols = jax.lax.broadcasted_iota(jnp.int32, (TB, E), 1)
    ce = jnp.sum(jnp.where(cols == e, comb_ref[rows, :], 0.0), axis=1,
                 keepdims=True)
    acc_ref[rows, :] = acc_ref[rows, :] + eo * ce

    @pl.when(e == E - 1)
    def _fin():
        o_ref[...] = acc_ref[rows, :]


def _moe_dense(x, combt, eg, eu, ed, sg, su, sd):
    grid = (E, NBLK)
    return pl.pallas_call(
        _moe_body,
        grid=grid,
        in_specs=[
            pl.BlockSpec((T, D), lambda e, i: (0, 0)),
            pl.BlockSpec((E, T), lambda e, i: (0, 0)),
            pl.BlockSpec((1, FF, D), lambda e, i: (e, 0, 0)),
            pl.BlockSpec((1, FF, D), lambda e, i: (e, 0, 0)),
            pl.BlockSpec((1, D, FF), lambda e, i: (e, 0, 0)),
            pl.BlockSpec((FF, D), lambda e, i: (0, 0)),
            pl.BlockSpec((FF, D), lambda e, i: (0, 0)),
            pl.BlockSpec((D, FF), lambda e, i: (0, 0)),
        ],
        out_specs=pl.BlockSpec(
            (TB, D), lambda e, i: (jnp.where(e == E - 1, i, 0), 0)),
        out_shape=jax.ShapeDtypeStruct((T, D), jnp.float32),
        scratch_shapes=[
            pltpu.VMEM((T, D), jnp.float32),
            pltpu.VMEM((T, D), jnp.bfloat16),
            pltpu.VMEM((T, E), jnp.float32),
        ],
        compiler_params=pltpu.CompilerParams(
            dimension_semantics=("arbitrary", "arbitrary"),
        ),
    )(x, combt, eg, eu, ed, sg, su, sd)


@jax.jit
def _moe(x, gate_weight, bias, eg, eu, ed, sg, su, sd):
    logits = _logits(gate_weight, x)
    b16 = jnp.broadcast_to(bias.reshape(E, 1), (E, 16)).reshape(E * 16)
    combt = _route(logits.reshape(-1), b16)
    return _moe_dense(x, combt.reshape(E, T), eg, eu, ed, sg, su, sd)


def kernel(hidden_states, gate_weight, e_score_correction_bias,
           expert_gate_w, expert_up_w, expert_down_w,
           shared_gate_w, shared_up_w, shared_down_w):
    orig_shape = hidden_states.shape
    x = hidden_states.reshape(-1, D).astype(jnp.float32)
    out = _moe(x, gate_weight, e_score_correction_bias,
               expert_gate_w, expert_up_w, expert_down_w,
               shared_gate_w, shared_up_w, shared_down_w)
    return out.reshape(orig_shape)
